# Initial kernel scaffold; baseline (speedup 1.0000x reference)
import numpy as np
import jax
import jax.numpy as jnp
from jax.experimental import pallas as pl
from jax.experimental.pallas import tpu as pltpu

_TB = 64
_HP1 = 40
_HP2 = 20
_THRESH = 0.5


def _body(x_ref, w1_ref, b1_ref, w2_ref, b2_ref, wf_ref, bf_ref,
          wt_ref, bt_ref, o_ref, xp_ref):
    tb = _TB
    r1 = tb * _HP1 + 4
    r2 = tb * _HP2

    xp_ref[...] = jnp.zeros(xp_ref.shape, jnp.float32)
    for im in range(tb):
        xp_ref[im * _HP1 + 3: im * _HP1 + 39, :] = x_ref[im * 36:(im + 1) * 36, :]
    xx = xp_ref[...]

    x3 = jnp.concatenate([xx[0:r1, :], xx[1:r1 + 1, :], xx[2:r1 + 2, :]], axis=1)
    y1 = jnp.dot(x3, w1_ref[...], preferred_element_type=jnp.float32)
    s1 = jnp.where(y1 + b1_ref[...] > _THRESH, 1.0, 0.0)
    wp1 = s1[:, 0::2] + s1[:, 1::2]
    hp1 = wp1[0::2, :] + wp1[1::2, :]
    hp1 = hp1[0:r2 + 2, :]
    v = jax.lax.broadcasted_iota(jnp.int32, (r2 + 2, 1), 0)
    keep = ((v % _HP2) >= 1) & ((v % _HP2) <= 18) & (v < r2)
    z1 = jnp.where(keep, hp1, 0.0)

    y2 = jnp.dot(z1[0:r2, :], w2_ref[0:180, :], preferred_element_type=jnp.float32)
    y2 = y2 + jnp.dot(z1[1:r2 + 1, :], w2_ref[180:360, :], preferred_element_type=jnp.float32)
    y2 = y2 + jnp.dot(z1[2:r2 + 2, :], w2_ref[360:540, :], preferred_element_type=jnp.float32)
    s2 = jnp.where(y2 + b2_ref[...] > _THRESH, 1.0, 0.0)
    wp2 = s2[:, 0::2] + s2[:, 1::2]
    hp2 = wp2[0::2, :] + wp2[1::2, :]

    flat = hp2.reshape(tb, 900)
    h = jnp.dot(flat, wf_ref[...], preferred_element_type=jnp.float32) + bf_ref[...]
    o_ref[...] = jnp.dot(h, wt_ref[...], preferred_element_type=jnp.float32) + bt_ref[...]


def _band(kw, wout):
    wp = np.arange(wout)
    e = (wp[None, :, None] == (wp[None, None, :] + np.arange(kw)[:, None, None] - 1))
    return jnp.asarray(e.astype(np.float32))


def kernel(conv1_w, conv1_b, conv2_w, conv2_b, fc_w, fc_b, task_w, task_b, x):
    b = x.shape[0]
    n_tasks = task_w.shape[0]
    tb = _TB
    assert b % tb == 0, b

    e1 = _band(3, 36)
    w1 = jnp.einsum('xuw,cdx->ducw', e1, conv1_w[:, 0]).reshape(108, 360)
    b1 = jnp.repeat(conv1_b, 36).reshape(1, 360)
    e2 = _band(3, 18)
    w2 = 0.25 * jnp.einsum('xuw,cidx->diucw', e2, conv2_w).reshape(540, 180)
    b2 = jnp.repeat(conv2_b, 18).reshape(1, 180)
    hh, cc, ww = np.meshgrid(np.arange(9), np.arange(10), np.arange(9), indexing='ij')
    perm = (cc * 81 + hh * 9 + ww).reshape(-1)
    wf = jnp.concatenate([0.25 * fc_w.T[perm, :], jnp.zeros((90, 50), jnp.float32)])
    bf = fc_b.reshape(1, 50)
    wt = jnp.transpose(task_w, (2, 0, 1)).reshape(50, n_tasks * 10)
    bt = task_b.reshape(1, n_tasks * 10)

    x2d = x.reshape(b * 36, 36)
    out = pl.pallas_call(
        _body,
        out_shape=jax.ShapeDtypeStruct((b, n_tasks * 10), jnp.float32),
        grid=(b // tb,),
        in_specs=[
            pl.BlockSpec((tb * 36, 36), lambda i: (i, 0)),
            pl.BlockSpec((108, 360), lambda i: (0, 0)),
            pl.BlockSpec((1, 360), lambda i: (0, 0)),
            pl.BlockSpec((540, 180), lambda i: (0, 0)),
            pl.BlockSpec((1, 180), lambda i: (0, 0)),
            pl.BlockSpec((900, 50), lambda i: (0, 0)),
            pl.BlockSpec((1, 50), lambda i: (0, 0)),
            pl.BlockSpec((50, n_tasks * 10), lambda i: (0, 0)),
            pl.BlockSpec((1, n_tasks * 10), lambda i: (0, 0)),
        ],
        out_specs=pl.BlockSpec((tb, n_tasks * 10), lambda i: (i, 0)),
        scratch_shapes=[pltpu.VMEM((tb * _HP1 + 8, 36), jnp.float32)],
        compiler_params=pltpu.CompilerParams(dimension_semantics=("parallel",)),
    )(x2d, w1, b1, w2, b2, wf, bf, wt, bt)
    return out.reshape(b, n_tasks, 10)


# speedup vs baseline: 1.9931x; 1.9931x over previous
import numpy as np
import jax
import jax.numpy as jnp
from jax.experimental import pallas as pl
from jax.experimental.pallas import tpu as pltpu

_TB = 64
_BIG = 1e30


def _body(x_ref, w1_ref, t1_ref, w2_ref, t2_ref, wf_ref, bf_ref,
          wt_ref, bt_ref, o_ref):
    tb = _TB
    rq = tb * 10
    xall = x_ref[...].reshape(4 * rq, 36)

    def cc(p, q, r):
        return jnp.concatenate([p, q, r], axis=1)

    x3 = jnp.concatenate([
        cc(xall[0 * rq:1 * rq], xall[1 * rq:2 * rq], xall[2 * rq:3 * rq]),
        cc(xall[1 * rq:2 * rq], xall[2 * rq:3 * rq], xall[3 * rq:4 * rq]),
        cc(xall[2 * rq:3 * rq], xall[3 * rq:4 * rq], xall[1:rq + 1]),
        cc(xall[3 * rq:4 * rq], xall[1:rq + 1], xall[rq + 1:2 * rq + 1]),
    ], axis=0)
    y1 = jnp.dot(x3, w1_ref[...], preferred_element_type=jnp.float32)
    k1 = jnp.where(y1 > t1_ref[...], 1.0, 0.0)
    wp = k1[:, 0:256] + k1[:, 256:512]
    hpe = wp[0 * rq:1 * rq] + wp[1 * rq:2 * rq]
    hpo = wp[2 * rq:3 * rq] + wp[3 * rq:4 * rq]
    q = jax.lax.broadcasted_iota(jnp.int32, (rq, 1), 0)
    ze = jnp.where(q % 10 != 0, hpe, 0.0)
    zo = jnp.where(q % 10 != 9, hpo, 0.0)

    m = rq - 1
    ze1 = ze[1:rq]
    zo1 = zo[1:rq]
    wd0 = w2_ref[0:256, :]
    wd1 = w2_ref[256:512, :]
    wd2 = w2_ref[512:768, :]

    def conv2(p0, p1, p2):
        y = jnp.dot(p0, wd0, preferred_element_type=jnp.float32)
        y = y + jnp.dot(p1, wd1, preferred_element_type=jnp.float32)
        y = y + jnp.dot(p2, wd2, preferred_element_type=jnp.float32)
        k = jnp.where(y > t2_ref[...], 1.0, 0.0)
        return k[:, 0:128] + k[:, 128:256]

    wpe = conv2(ze[0:m], zo[0:m], ze1)
    wpo = conv2(zo[0:m], ze1, zo1)
    hp2 = jnp.pad(wpe + wpo, ((0, 1), (0, 0)))

    flat = hp2.reshape(tb, 1280)
    h = jnp.dot(flat, wf_ref[...], preferred_element_type=jnp.float32) + bf_ref[...]
    o_ref[...] = jnp.dot(h, wt_ref[...], preferred_element_type=jnp.float32) + bt_ref[...]


def _band(win, wout):
    u = np.arange(win)[:, None, None, None]
    pw = np.arange(2)[None, :, None, None]
    w2 = np.arange(wout)[None, None, :, None]
    dx = np.arange(3)[None, None, None, :]
    return jnp.asarray((u == 2 * w2 + pw + dx - 1).astype(np.float32))


def _padlane(a, n):
    return jnp.pad(a, [(0, 0)] * (a.ndim - 1) + [(0, n - a.shape[-1])])


def kernel(conv1_w, conv1_b, conv2_w, conv2_b, fc_w, fc_b, task_w, task_b, x):
    b = x.shape[0]
    n_tasks = task_w.shape[0]
    tb = _TB
    assert b % tb == 0, b

    e1 = _band(36, 18)
    w1 = jnp.einsum('upwx,cdx->dupcw', e1, conv1_w[:, 0])
    w1 = _padlane(w1.reshape(108, 2, 180), 256).reshape(108, 512)
    t1 = _padlane(jnp.tile(jnp.repeat(0.5 - conv1_b, 18), 2).reshape(2, 180), 256)
    t1 = (t1 + _BIG * (jnp.arange(256) >= 180)).reshape(1, 512)
    e2 = _band(18, 9)
    w2 = 0.25 * jnp.einsum('upwx,cidx->diupcw', e2, conv2_w)
    w2 = _padlane(w2.reshape(3, 180, 2, 90), 128)
    w2 = jnp.pad(w2, ((0, 0), (0, 76), (0, 0), (0, 0))).reshape(768, 256)
    t2 = _padlane(jnp.tile(jnp.repeat(0.5 - conv2_b, 9), 2).reshape(2, 90), 128)
    t2 = (t2 + _BIG * (jnp.arange(128) >= 90)).reshape(1, 256)
    hh, cc_, ww = np.meshgrid(np.arange(9), np.arange(10), np.arange(9), indexing='ij')
    perm = (cc_ * 81 + hh * 9 + ww).reshape(-1)
    wf = jnp.concatenate([0.25 * fc_w.T[perm, :], jnp.zeros((90, 50), jnp.float32)])
    wf = jnp.pad(wf.reshape(10, 90, 50), ((0, 0), (0, 38), (0, 0))).reshape(1280, 50)
    bf = fc_b.reshape(1, 50)
    wt = jnp.transpose(task_w, (2, 0, 1)).reshape(50, n_tasks * 10)
    bt = task_b.reshape(1, n_tasks * 10)

    xe = x.reshape(b, 9, 4, 36)
    xs = jnp.stack([
        jnp.pad(xe[:, :, 1, :], ((0, 0), (1, 0), (0, 0))).reshape(b * 10, 36),
        jnp.pad(xe[:, :, 2, :], ((0, 0), (1, 0), (0, 0))).reshape(b * 10, 36),
        jnp.pad(xe[:, :, 3, :], ((0, 0), (1, 0), (0, 0))).reshape(b * 10, 36),
        jnp.pad(xe[:, :, 0, :], ((0, 0), (0, 1), (0, 0))).reshape(b * 10, 36),
    ])

    out = pl.pallas_call(
        _body,
        out_shape=jax.ShapeDtypeStruct((b, n_tasks * 10), jnp.float32),
        grid=(b // tb,),
        in_specs=[
            pl.BlockSpec((4, tb * 10, 36), lambda i: (0, i, 0)),
            pl.BlockSpec((108, 512), lambda i: (0, 0)),
            pl.BlockSpec((1, 512), lambda i: (0, 0)),
            pl.BlockSpec((768, 256), lambda i: (0, 0)),
            pl.BlockSpec((1, 256), lambda i: (0, 0)),
            pl.BlockSpec((1280, 50), lambda i: (0, 0)),
            pl.BlockSpec((1, 50), lambda i: (0, 0)),
            pl.BlockSpec((50, n_tasks * 10), lambda i: (0, 0)),
            pl.BlockSpec((1, n_tasks * 10), lambda i: (0, 0)),
        ],
        out_specs=pl.BlockSpec((tb, n_tasks * 10), lambda i: (i, 0)),
        compiler_params=pltpu.CompilerParams(dimension_semantics=("parallel",)),
    )(xs, w1, t1, w2, t2, wf, bf, wt, bt)
    return out.reshape(b, n_tasks, 10)


# speedup vs baseline: 2.0503x; 1.0287x over previous
import numpy as np
import jax
import jax.numpy as jnp
from jax.experimental import pallas as pl
from jax.experimental.pallas import tpu as pltpu

_TB = 128
_BIG = 1e30


def _body(x_ref, w1_ref, t1_ref, w2_ref, t2_ref, wf_ref, bf_ref,
          wt_ref, bt_ref, o_ref):
    tb = _TB
    rq = tb * 10
    xall = x_ref[...].reshape(4 * rq, 36)

    def cc(p, q, r):
        return jnp.concatenate([p, q, r], axis=1)

    x3 = jnp.concatenate([
        cc(xall[0 * rq:1 * rq], xall[1 * rq:2 * rq], xall[2 * rq:3 * rq]),
        cc(xall[1 * rq:2 * rq], xall[2 * rq:3 * rq], xall[3 * rq:4 * rq]),
        cc(xall[2 * rq:3 * rq], xall[3 * rq:4 * rq], xall[1:rq + 1]),
        cc(xall[3 * rq:4 * rq], xall[1:rq + 1], xall[rq + 1:2 * rq + 1]),
    ], axis=0)
    y1 = jnp.dot(x3, w1_ref[...], preferred_element_type=jnp.float32)
    k1 = jnp.where(y1 > t1_ref[...], 1.0, 0.0)
    wp = k1[:, 0:256] + k1[:, 256:512]
    hpe = wp[0 * rq:1 * rq] + wp[1 * rq:2 * rq]
    hpo = wp[2 * rq:3 * rq] + wp[3 * rq:4 * rq]
    q = jax.lax.broadcasted_iota(jnp.int32, (rq, 1), 0)
    ze = jnp.where(q % 10 != 0, hpe, 0.0)
    zo = jnp.where(q % 10 != 9, hpo, 0.0)

    m = rq - 1
    ze1 = ze[1:rq]
    zo1 = zo[1:rq]
    wd0 = w2_ref[0:256, :]
    wd1 = w2_ref[256:512, :]
    wd2 = w2_ref[512:768, :]

    def conv2(p0, p1, p2):
        y = jnp.dot(p0, wd0, preferred_element_type=jnp.float32)
        y = y + jnp.dot(p1, wd1, preferred_element_type=jnp.float32)
        y = y + jnp.dot(p2, wd2, preferred_element_type=jnp.float32)
        k = jnp.where(y > t2_ref[...], 1.0, 0.0)
        return k[:, 0:128] + k[:, 128:256]

    wpe = conv2(ze[0:m], zo[0:m], ze1)
    wpo = conv2(zo[0:m], ze1, zo1)
    hp2 = jnp.pad(wpe + wpo, ((0, 1), (0, 0)))

    flat = hp2.reshape(tb, 1280)
    h = jnp.dot(flat, wf_ref[...], preferred_element_type=jnp.float32) + bf_ref[...]
    o_ref[...] = jnp.dot(h, wt_ref[...], preferred_element_type=jnp.float32) + bt_ref[...]


def _band(win, wout):
    u = np.arange(win)[:, None, None, None]
    pw = np.arange(2)[None, :, None, None]
    w2 = np.arange(wout)[None, None, :, None]
    dx = np.arange(3)[None, None, None, :]
    return jnp.asarray((u == 2 * w2 + pw + dx - 1).astype(np.float32))


def _padlane(a, n):
    return jnp.pad(a, [(0, 0)] * (a.ndim - 1) + [(0, n - a.shape[-1])])


def kernel(conv1_w, conv1_b, conv2_w, conv2_b, fc_w, fc_b, task_w, task_b, x):
    b = x.shape[0]
    n_tasks = task_w.shape[0]
    tb = _TB
    assert b % tb == 0, b

    e1 = _band(36, 18)
    w1 = jnp.einsum('upwx,cdx->dupcw', e1, conv1_w[:, 0])
    w1 = _padlane(w1.reshape(108, 2, 180), 256).reshape(108, 512)
    t1 = _padlane(jnp.tile(jnp.repeat(0.5 - conv1_b, 18), 2).reshape(2, 180), 256)
    t1 = (t1 + _BIG * (jnp.arange(256) >= 180)).reshape(1, 512)
    e2 = _band(18, 9)
    w2 = 0.25 * jnp.einsum('upwx,cidx->diupcw', e2, conv2_w)
    w2 = _padlane(w2.reshape(3, 180, 2, 90), 128)
    w2 = jnp.pad(w2, ((0, 0), (0, 76), (0, 0), (0, 0))).reshape(768, 256)
    t2 = _padlane(jnp.tile(jnp.repeat(0.5 - conv2_b, 9), 2).reshape(2, 90), 128)
    t2 = (t2 + _BIG * (jnp.arange(128) >= 90)).reshape(1, 256)
    hh, cc_, ww = np.meshgrid(np.arange(9), np.arange(10), np.arange(9), indexing='ij')
    perm = (cc_ * 81 + hh * 9 + ww).reshape(-1)
    wf = jnp.concatenate([0.25 * fc_w.T[perm, :], jnp.zeros((90, 50), jnp.float32)])
    wf = jnp.pad(wf.reshape(10, 90, 50), ((0, 0), (0, 38), (0, 0))).reshape(1280, 50)
    bf = fc_b.reshape(1, 50)
    wt = jnp.transpose(task_w, (2, 0, 1)).reshape(50, n_tasks * 10)
    bt = task_b.reshape(1, n_tasks * 10)

    xe = x.reshape(b, 9, 4, 36)
    xs = jnp.stack([
        jnp.pad(xe[:, :, 1, :], ((0, 0), (1, 0), (0, 0))).reshape(b * 10, 36),
        jnp.pad(xe[:, :, 2, :], ((0, 0), (1, 0), (0, 0))).reshape(b * 10, 36),
        jnp.pad(xe[:, :, 3, :], ((0, 0), (1, 0), (0, 0))).reshape(b * 10, 36),
        jnp.pad(xe[:, :, 0, :], ((0, 0), (0, 1), (0, 0))).reshape(b * 10, 36),
    ])

    out = pl.pallas_call(
        _body,
        out_shape=jax.ShapeDtypeStruct((b, n_tasks * 10), jnp.float32),
        grid=(b // tb,),
        in_specs=[
            pl.BlockSpec((4, tb * 10, 36), lambda i: (0, i, 0)),
            pl.BlockSpec((108, 512), lambda i: (0, 0)),
            pl.BlockSpec((1, 512), lambda i: (0, 0)),
            pl.BlockSpec((768, 256), lambda i: (0, 0)),
            pl.BlockSpec((1, 256), lambda i: (0, 0)),
            pl.BlockSpec((1280, 50), lambda i: (0, 0)),
            pl.BlockSpec((1, 50), lambda i: (0, 0)),
            pl.BlockSpec((50, n_tasks * 10), lambda i: (0, 0)),
            pl.BlockSpec((1, n_tasks * 10), lambda i: (0, 0)),
        ],
        out_specs=pl.BlockSpec((tb, n_tasks * 10), lambda i: (i, 0)),
        compiler_params=pltpu.CompilerParams(dimension_semantics=("parallel",)),
    )(xs, w1, t1, w2, t2, wf, bf, wt, bt)
    return out.reshape(b, n_tasks, 10)


# speedup vs baseline: 3.4898x; 1.7021x over previous
import numpy as np
import jax
import jax.numpy as jnp
from jax.experimental import pallas as pl
from jax.experimental.pallas import tpu as pltpu

_TB = 128
_BIG = 1e30


def _body(x_ref, w1_ref, t1_ref, w2_ref, t2_ref, wf_ref, bf_ref,
          wt_ref, bt_ref, o_ref):
    tb = _TB
    rq = tb * 10
    xall = x_ref[...].reshape(4 * rq, 36)

    def cc(p, q, r):
        return jnp.concatenate([p, q, r], axis=1)

    x3 = jnp.concatenate([
        cc(xall[1 * rq:2 * rq], xall[2 * rq:3 * rq], xall[3 * rq:4 * rq]),
        cc(xall[2 * rq:3 * rq], xall[3 * rq:4 * rq], xall[1:rq + 1]),
        cc(xall[3 * rq:4 * rq], xall[1:rq + 1], xall[rq + 1:2 * rq + 1]),
        cc(xall[1:rq + 1], xall[rq + 1:2 * rq + 1], xall[2 * rq + 1:3 * rq + 1]),
    ], axis=0)
    y1 = jnp.dot(x3, w1_ref[...], preferred_element_type=jnp.float32)
    k1 = jnp.where(y1 > t1_ref[...], 1.0, 0.0)
    wp = k1[:, 0:256] + k1[:, 256:512]
    hpe = wp[0 * rq:1 * rq] + wp[1 * rq:2 * rq]
    hpo = wp[2 * rq:3 * rq] + wp[3 * rq:4 * rq]
    q = jax.lax.broadcasted_iota(jnp.int32, (rq, 1), 0)
    ze = jnp.where(q % 10 != 0, hpe, 0.0)
    zo = jnp.where(q % 10 != 9, hpo, 0.0)

    m = rq - 1
    ze1 = ze[1:rq]
    zo1 = zo[1:rq]
    wd0 = w2_ref[0:256, :]
    wd1 = w2_ref[256:512, :]
    wd2 = w2_ref[512:768, :]

    def conv2(p0, p1, p2):
        y = jnp.dot(p0, wd0, preferred_element_type=jnp.float32)
        y = y + jnp.dot(p1, wd1, preferred_element_type=jnp.float32)
        y = y + jnp.dot(p2, wd2, preferred_element_type=jnp.float32)
        k = jnp.where(y > t2_ref[...], 1.0, 0.0)
        return k[:, 0:128] + k[:, 128:256]

    wpe = conv2(ze[0:m], zo[0:m], ze1)
    wpo = conv2(zo[0:m], ze1, zo1)
    hp2 = jnp.pad(wpe + wpo, ((0, 1), (0, 0)))

    flat = hp2.reshape(tb, 1280)
    h = jnp.dot(flat, wf_ref[...], preferred_element_type=jnp.float32) + bf_ref[...]
    o_ref[...] = jnp.dot(h, wt_ref[...], preferred_element_type=jnp.float32) + bt_ref[...]


def _band(win, wout):
    u = np.arange(win)[:, None, None, None]
    pw = np.arange(2)[None, :, None, None]
    w2 = np.arange(wout)[None, None, :, None]
    dx = np.arange(3)[None, None, None, :]
    return jnp.asarray((u == 2 * w2 + pw + dx - 1).astype(np.float32))


def _padlane(a, n):
    return jnp.pad(a, [(0, 0)] * (a.ndim - 1) + [(0, n - a.shape[-1])])


def kernel(conv1_w, conv1_b, conv2_w, conv2_b, fc_w, fc_b, task_w, task_b, x):
    b = x.shape[0]
    n_tasks = task_w.shape[0]
    tb = _TB
    assert b % tb == 0, b

    e1 = _band(36, 18)
    w1 = jnp.einsum('upwx,cdx->dupcw', e1, conv1_w[:, 0])
    w1 = _padlane(w1.reshape(108, 2, 180), 256).reshape(108, 512)
    t1 = _padlane(jnp.tile(jnp.repeat(0.5 - conv1_b, 18), 2).reshape(2, 180), 256)
    t1 = (t1 + _BIG * (jnp.arange(256) >= 180)).reshape(1, 512)
    e2 = _band(18, 9)
    w2 = 0.25 * jnp.einsum('upwx,cidx->diupcw', e2, conv2_w)
    w2 = _padlane(w2.reshape(3, 180, 2, 90), 128)
    w2 = jnp.pad(w2, ((0, 0), (0, 76), (0, 0), (0, 0))).reshape(768, 256)
    t2 = _padlane(jnp.tile(jnp.repeat(0.5 - conv2_b, 9), 2).reshape(2, 90), 128)
    t2 = (t2 + _BIG * (jnp.arange(128) >= 90)).reshape(1, 256)
    hh, cc_, ww = np.meshgrid(np.arange(9), np.arange(10), np.arange(9), indexing='ij')
    perm = (cc_ * 81 + hh * 9 + ww).reshape(-1)
    wf = jnp.concatenate([0.25 * fc_w.T[perm, :], jnp.zeros((90, 50), jnp.float32)])
    wf = jnp.pad(wf.reshape(10, 90, 50), ((0, 0), (0, 38), (0, 0))).reshape(1280, 50)
    bf = fc_b.reshape(1, 50)
    wt = jnp.transpose(task_w, (2, 0, 1)).reshape(50, n_tasks * 10)
    bt = task_b.reshape(1, n_tasks * 10)

    xt = jnp.transpose(x.reshape(b, 9, 4, 36), (2, 0, 1, 3))
    xs = jnp.pad(xt, ((0, 0), (0, 0), (1, 0), (0, 0))).reshape(4, b * 10, 36)

    out = pl.pallas_call(
        _body,
        out_shape=jax.ShapeDtypeStruct((b, n_tasks * 10), jnp.float32),
        grid=(b // tb,),
        in_specs=[
            pl.BlockSpec((4, tb * 10, 36), lambda i: (0, i, 0)),
            pl.BlockSpec((108, 512), lambda i: (0, 0)),
            pl.BlockSpec((1, 512), lambda i: (0, 0)),
            pl.BlockSpec((768, 256), lambda i: (0, 0)),
            pl.BlockSpec((1, 256), lambda i: (0, 0)),
            pl.BlockSpec((1280, 50), lambda i: (0, 0)),
            pl.BlockSpec((1, 50), lambda i: (0, 0)),
            pl.BlockSpec((50, n_tasks * 10), lambda i: (0, 0)),
            pl.BlockSpec((1, n_tasks * 10), lambda i: (0, 0)),
        ],
        out_specs=pl.BlockSpec((tb, n_tasks * 10), lambda i: (i, 0)),
        compiler_params=pltpu.CompilerParams(dimension_semantics=("parallel",)),
    )(xs, w1, t1, w2, t2, wf, bf, wt, bt)
    return out.reshape(b, n_tasks, 10)
